# baseline (speedup 1.0000x reference)
import jax
import jax.numpy as jnp
from jax.experimental import pallas as pl
from jax.experimental.pallas import tpu as pltpu

_B, _A, _S, _C = 64, 3, 52, 11
_NCH = 5 + _C
_N = _B * _A * _S * _S
_LANES = 128
_ROWS = _N // _LANES
_RBLK = 104
_GRID = _ROWS // _RBLK


def _floordiv_f32(nf, d):
    return jnp.floor((nf + 0.5) * (1.0 / d))


def _yolo_kernel(anchor_ref, p_ref, t_ref, out_ref, acc_ref):
    g = pl.program_id(0)

    @pl.when(g == 0)
    def _init():
        acc_ref[...] = jnp.zeros_like(acc_ref)

    i = jax.lax.broadcasted_iota(jnp.int32, (_RBLK, _LANES), 0).astype(jnp.float32)
    j = jax.lax.broadcasted_iota(jnp.int32, (_RBLK, _LANES), 1).astype(jnp.float32)
    nf = jnp.float32(_RBLK * _LANES) * g.astype(jnp.float32) + i * _LANES + j

    q1 = _floordiv_f32(nf, _S)
    gx = nf - _S * q1
    q2 = _floordiv_f32(q1, _S)
    gy = q1 - _S * q2
    q3 = _floordiv_f32(q2, _A)
    af = q2 - _A * q3

    is_a0 = af < 0.5
    is_a1 = af < 1.5
    aw = jnp.where(is_a0, anchor_ref[0, 0],
                   jnp.where(is_a1, anchor_ref[1, 0], anchor_ref[2, 0]))
    ah = jnp.where(is_a0, anchor_ref[0, 1],
                   jnp.where(is_a1, anchor_ref[1, 1], anchor_ref[2, 1]))

    po = p_ref[0].astype(jnp.float32)
    pxl = p_ref[1].astype(jnp.float32)
    pyl = p_ref[2].astype(jnp.float32)
    pw = p_ref[3].astype(jnp.float32)
    ph = p_ref[4].astype(jnp.float32)

    tobj = t_ref[0].astype(jnp.float32)
    tx = t_ref[1].astype(jnp.float32)
    ty = t_ref[2].astype(jnp.float32)
    tw = t_ref[3].astype(jnp.float32)
    th = t_ref[4].astype(jnp.float32)
    tcls = t_ref[5].astype(jnp.float32)

    obj_m = tobj == 1.0

    sp = jnp.maximum(po, 0.0) + jnp.log1p(jnp.exp(-jnp.abs(po)))

    px = jax.nn.sigmoid(pxl)
    py = jax.nn.sigmoid(pyl)

    ix = gx + px
    iy = gy + py
    iw = aw * jnp.exp(pw)
    ih = ah * jnp.exp(ph)
    b1x1 = ix - 0.5 * iw
    b1x2 = ix + 0.5 * iw
    b1y1 = iy - 0.5 * ih
    b1y2 = iy + 0.5 * ih
    b2x1 = tx - 0.5 * tw
    b2x2 = tx + 0.5 * tw
    b2y1 = ty - 0.5 * th
    b2y2 = ty + 0.5 * th
    interw = jnp.clip(jnp.minimum(b1x2, b2x2) - jnp.maximum(b1x1, b2x1), 0.0)
    interh = jnp.clip(jnp.minimum(b1y2, b2y2) - jnp.maximum(b1y1, b2y1), 0.0)
    inter = interw * interh
    area1 = jnp.abs(iw * ih)
    area2 = jnp.abs(tw * th)
    iou = inter / (area1 + area2 - inter + 1e-6)

    obj_bce = sp - po * iou

    tbx = tx - gx
    tby = ty - gy
    tbw = jnp.log(1e-16 + tw / aw)
    tbh = jnp.log(1e-16 + th / ah)
    dx = px - tbx
    dy = py - tby
    dw = pw - tbw
    dh = ph - tbh
    box_sq = dx * dx + dy * dy + dw * dw + dh * dh

    l0 = p_ref[5].astype(jnp.float32)
    mx = l0
    for k in range(6, 5 + _C):
        mx = jnp.maximum(mx, p_ref[k].astype(jnp.float32))
    ssum = jnp.exp(l0 - mx)
    picked = jnp.where(tcls == 0.0, l0, 0.0)
    for k in range(1, _C):
        lk = p_ref[5 + k].astype(jnp.float32)
        ssum = ssum + jnp.exp(lk - mx)
        picked = picked + jnp.where(tcls == jnp.float32(k), lk, 0.0)
    cls_term = mx + jnp.log(ssum) - picked

    term_a = jnp.where(obj_m, 2.5 * box_sq + obj_bce + cls_term, 0.0)
    term_b = jnp.where(obj_m, 0.0, sp)

    acc_ref[0, :, :] = acc_ref[0, :, :] + term_a
    acc_ref[1, :, :] = acc_ref[1, :, :] + term_b
    acc_ref[2, :, :] = acc_ref[2, :, :] + obj_m.astype(jnp.float32)

    @pl.when(g == _GRID - 1)
    def _fini():
        s_a = jnp.sum(acc_ref[0, :, :])
        s_b = jnp.sum(acc_ref[1, :, :])
        n_obj = jnp.sum(acc_ref[2, :, :])
        out_ref[0, 0] = s_a / n_obj + 10.0 * s_b / (jnp.float32(_N) - n_obj)


def kernel(predictions, target, anchor_sizes):
    pb = predictions.astype(jnp.bfloat16)
    tb = target.astype(jnp.bfloat16)
    pt = jnp.moveaxis(pb, 4, 0).reshape(_NCH, _ROWS, _LANES)
    tt = jnp.moveaxis(tb, 4, 0).reshape(6, _ROWS, _LANES)
    out = pl.pallas_call(
        _yolo_kernel,
        grid=(_GRID,),
        in_specs=[
            pl.BlockSpec(memory_space=pltpu.SMEM),
            pl.BlockSpec((_NCH, _RBLK, _LANES), lambda g: (0, g, 0)),
            pl.BlockSpec((6, _RBLK, _LANES), lambda g: (0, g, 0)),
        ],
        out_specs=pl.BlockSpec(memory_space=pltpu.SMEM),
        out_shape=jax.ShapeDtypeStruct((1, 1), jnp.float32),
        scratch_shapes=[pltpu.VMEM((3, _RBLK, _LANES), jnp.float32)],
    )(anchor_sizes, pt, tt)
    return out[0, 0]


# speedup vs baseline: 6.5272x; 6.5272x over previous
import jax
import jax.numpy as jnp
from jax.experimental import pallas as pl
from jax.experimental.pallas import tpu as pltpu

_B, _A, _S, _C = 64, 3, 52, 11
_NCH = 5 + _C
_N = _B * _A * _S * _S
_ROWS = _B * _A * _S
_RBLK = 208
_GRID = _ROWS // _RBLK


def _floordiv_f32(nf, d):
    return jnp.floor((nf + 0.5) * (1.0 / d))


def _yolo_kernel(anchor_ref, p_ref, t_ref, out_ref, acc_ref):
    g = pl.program_id(0)

    @pl.when(g == 0)
    def _init():
        acc_ref[...] = jnp.zeros_like(acc_ref)

    i = jax.lax.broadcasted_iota(jnp.int32, (_RBLK, _S), 0).astype(jnp.float32)
    gx = jax.lax.broadcasted_iota(jnp.int32, (_RBLK, _S), 1).astype(jnp.float32)
    rf = jnp.float32(_RBLK) * g.astype(jnp.float32) + i

    q1 = _floordiv_f32(rf, _S)
    gy = rf - _S * q1
    q2 = _floordiv_f32(q1, _A)
    af = q1 - _A * q2

    is_a0 = af < 0.5
    is_a1 = af < 1.5
    aw = jnp.where(is_a0, anchor_ref[0, 0],
                   jnp.where(is_a1, anchor_ref[1, 0], anchor_ref[2, 0]))
    ah = jnp.where(is_a0, anchor_ref[0, 1],
                   jnp.where(is_a1, anchor_ref[1, 1], anchor_ref[2, 1]))

    po = p_ref[0]
    pxl = p_ref[1]
    pyl = p_ref[2]
    pw = p_ref[3]
    ph = p_ref[4]

    tobj = t_ref[0]
    tx = t_ref[1]
    ty = t_ref[2]
    tw = t_ref[3]
    th = t_ref[4]
    tcls = t_ref[5]

    obj_m = tobj == 1.0

    sp = jnp.maximum(po, 0.0) + jnp.log1p(jnp.exp(-jnp.abs(po)))

    px = jax.nn.sigmoid(pxl)
    py = jax.nn.sigmoid(pyl)

    ix = gx + px
    iy = gy + py
    iw = aw * jnp.exp(pw)
    ih = ah * jnp.exp(ph)
    b1x1 = ix - 0.5 * iw
    b1x2 = ix + 0.5 * iw
    b1y1 = iy - 0.5 * ih
    b1y2 = iy + 0.5 * ih
    b2x1 = tx - 0.5 * tw
    b2x2 = tx + 0.5 * tw
    b2y1 = ty - 0.5 * th
    b2y2 = ty + 0.5 * th
    interw = jnp.clip(jnp.minimum(b1x2, b2x2) - jnp.maximum(b1x1, b2x1), 0.0)
    interh = jnp.clip(jnp.minimum(b1y2, b2y2) - jnp.maximum(b1y1, b2y1), 0.0)
    inter = interw * interh
    area1 = jnp.abs(iw * ih)
    area2 = jnp.abs(tw * th)
    iou = inter / (area1 + area2 - inter + 1e-6)

    obj_bce = sp - po * iou

    tbx = tx - gx
    tby = ty - gy
    tbw = jnp.log(1e-16 + tw / aw)
    tbh = jnp.log(1e-16 + th / ah)
    dx = px - tbx
    dy = py - tby
    dw = pw - tbw
    dh = ph - tbh
    box_sq = dx * dx + dy * dy + dw * dw + dh * dh

    l0 = p_ref[5]
    mx = l0
    for k in range(6, 5 + _C):
        mx = jnp.maximum(mx, p_ref[k])
    ssum = jnp.exp(l0 - mx)
    picked = jnp.where(tcls == 0.0, l0, 0.0)
    for k in range(1, _C):
        lk = p_ref[5 + k]
        ssum = ssum + jnp.exp(lk - mx)
        picked = picked + jnp.where(tcls == jnp.float32(k), lk, 0.0)
    cls_term = mx + jnp.log(ssum) - picked

    term_a = jnp.where(obj_m, 2.5 * box_sq + obj_bce + cls_term, 0.0)
    term_b = jnp.where(obj_m, 0.0, sp)

    acc_ref[0, :, :] = acc_ref[0, :, :] + term_a
    acc_ref[1, :, :] = acc_ref[1, :, :] + term_b
    acc_ref[2, :, :] = acc_ref[2, :, :] + obj_m.astype(jnp.float32)

    @pl.when(g == _GRID - 1)
    def _fini():
        s_a = jnp.sum(acc_ref[0, :, :])
        s_b = jnp.sum(acc_ref[1, :, :])
        n_obj = jnp.sum(acc_ref[2, :, :])
        out_ref[0, 0] = s_a / n_obj + 10.0 * s_b / (jnp.float32(_N) - n_obj)


def kernel(predictions, target, anchor_sizes):
    pt = jnp.moveaxis(predictions, 4, 0).reshape(_NCH, _ROWS, _S)
    tt = jnp.moveaxis(target, 4, 0).reshape(6, _ROWS, _S)
    out = pl.pallas_call(
        _yolo_kernel,
        grid=(_GRID,),
        in_specs=[
            pl.BlockSpec(memory_space=pltpu.SMEM),
            pl.BlockSpec((_NCH, _RBLK, _S), lambda g: (0, g, 0)),
            pl.BlockSpec((6, _RBLK, _S), lambda g: (0, g, 0)),
        ],
        out_specs=pl.BlockSpec(memory_space=pltpu.SMEM),
        out_shape=jax.ShapeDtypeStruct((1, 1), jnp.float32),
        scratch_shapes=[pltpu.VMEM((3, _RBLK, _S), jnp.float32)],
    )(anchor_sizes, pt, tt)
    return out[0, 0]


# speedup vs baseline: 10.7625x; 1.6489x over previous
import jax
import jax.numpy as jnp
from jax.experimental import pallas as pl
from jax.experimental.pallas import tpu as pltpu

_B, _A, _S, _C = 64, 3, 52, 11
_NCH = 5 + _C
_N = _B * _A * _S * _S
_BBLK = 2
_GRID = _B // _BBLK


def _yolo_kernel(anchor_ref, p_ref, t_ref, out_ref, acc_ref):
    g = pl.program_id(0)

    @pl.when(g == 0)
    def _init():
        acc_ref[0, 0] = 0.0
        acc_ref[0, 1] = 0.0
        acc_ref[0, 2] = 0.0

    shape = (_BBLK, _A, _S, _S)
    gy = jax.lax.broadcasted_iota(jnp.int32, shape, 2).astype(jnp.float32)
    gx = jax.lax.broadcasted_iota(jnp.int32, shape, 3).astype(jnp.float32)
    ia = jax.lax.broadcasted_iota(jnp.int32, shape, 1)

    aw = jnp.where(ia == 0, anchor_ref[0, 0],
                   jnp.where(ia == 1, anchor_ref[1, 0], anchor_ref[2, 0]))
    ah = jnp.where(ia == 0, anchor_ref[0, 1],
                   jnp.where(ia == 1, anchor_ref[1, 1], anchor_ref[2, 1]))

    po = p_ref[0]
    pxl = p_ref[1]
    pyl = p_ref[2]
    pw = p_ref[3]
    ph = p_ref[4]

    tobj = t_ref[0]
    tx = t_ref[1]
    ty = t_ref[2]
    tw = t_ref[3]
    th = t_ref[4]
    tcls = t_ref[5]

    obj_m = tobj == 1.0

    sp = jnp.maximum(po, 0.0) + jnp.log1p(jnp.exp(-jnp.abs(po)))

    px = jax.nn.sigmoid(pxl)
    py = jax.nn.sigmoid(pyl)

    ix = gx + px
    iy = gy + py
    iw = aw * jnp.exp(pw)
    ih = ah * jnp.exp(ph)
    b1x1 = ix - 0.5 * iw
    b1x2 = ix + 0.5 * iw
    b1y1 = iy - 0.5 * ih
    b1y2 = iy + 0.5 * ih
    b2x1 = tx - 0.5 * tw
    b2x2 = tx + 0.5 * tw
    b2y1 = ty - 0.5 * th
    b2y2 = ty + 0.5 * th
    interw = jnp.clip(jnp.minimum(b1x2, b2x2) - jnp.maximum(b1x1, b2x1), 0.0)
    interh = jnp.clip(jnp.minimum(b1y2, b2y2) - jnp.maximum(b1y1, b2y1), 0.0)
    inter = interw * interh
    area1 = jnp.abs(iw * ih)
    area2 = jnp.abs(tw * th)
    iou = inter / (area1 + area2 - inter + 1e-6)

    obj_bce = sp - po * iou

    tbx = tx - gx
    tby = ty - gy
    tbw = jnp.log(1e-16 + tw / aw)
    tbh = jnp.log(1e-16 + th / ah)
    dx = px - tbx
    dy = py - tby
    dw = pw - tbw
    dh = ph - tbh
    box_sq = dx * dx + dy * dy + dw * dw + dh * dh

    l0 = p_ref[5]
    mx = l0
    for k in range(6, 5 + _C):
        mx = jnp.maximum(mx, p_ref[k])
    ssum = jnp.exp(l0 - mx)
    picked = jnp.where(tcls == 0.0, l0, 0.0)
    for k in range(1, _C):
        lk = p_ref[5 + k]
        ssum = ssum + jnp.exp(lk - mx)
        picked = picked + jnp.where(tcls == jnp.float32(k), lk, 0.0)
    cls_term = mx + jnp.log(ssum) - picked

    term_a = jnp.where(obj_m, 2.5 * box_sq + obj_bce + cls_term, 0.0)
    term_b = jnp.where(obj_m, 0.0, sp)

    acc_ref[0, 0] = acc_ref[0, 0] + jnp.sum(term_a)
    acc_ref[0, 1] = acc_ref[0, 1] + jnp.sum(term_b)
    acc_ref[0, 2] = acc_ref[0, 2] + jnp.sum(obj_m.astype(jnp.float32))

    @pl.when(g == _GRID - 1)
    def _fini():
        s_a = acc_ref[0, 0]
        s_b = acc_ref[0, 1]
        n_obj = acc_ref[0, 2]
        out_ref[0, 0] = s_a / n_obj + 10.0 * s_b / (jnp.float32(_N) - n_obj)


def kernel(predictions, target, anchor_sizes):
    pt = jnp.moveaxis(predictions, 4, 0)
    tt = jnp.moveaxis(target, 4, 0)
    out = pl.pallas_call(
        _yolo_kernel,
        grid=(_GRID,),
        in_specs=[
            pl.BlockSpec(memory_space=pltpu.SMEM),
            pl.BlockSpec((_NCH, _BBLK, _A, _S, _S), lambda g: (0, g, 0, 0, 0)),
            pl.BlockSpec((6, _BBLK, _A, _S, _S), lambda g: (0, g, 0, 0, 0)),
        ],
        out_specs=pl.BlockSpec(memory_space=pltpu.SMEM),
        out_shape=jax.ShapeDtypeStruct((1, 1), jnp.float32),
        scratch_shapes=[pltpu.SMEM((1, 3), jnp.float32)],
    )(anchor_sizes, pt, tt)
    return out[0, 0]
